# baseline (device time: 153024 ns/iter reference)
import jax
import jax.numpy as jnp
from jax import lax
from jax.experimental import pallas as pl
from jax.experimental.pallas import tpu as pltpu

N_DEV = 4


def kernel(t):
    m, n = t.shape
    half = m // 4
    quar = m // 8
    breg = m // 2

    def f(s):
        r = jnp.maximum(s, 0.0)
        return jnp.tanh(s) * s * s + r * r * r

    def body(t_ref, out_ref, bufA1, bufB1, bufA2, bufB2, tA_loc, tB_loc,
             fA, fB, ssems, rsems, lsems):
        p = lax.axis_index("i")
        xp = p ^ 1
        yp = 3 - p

        h1A = jnp.where((p == 1) | (p == 2), 1, 0)
        q2A = jnp.where(p >= 2, 1, 0)
        h1B = q2A
        q2B = p % 2
        gA = (2 * h1A + q2A) * quar
        gB = breg + (2 * h1B + q2B) * quar

        cpA = pltpu.make_async_copy(
            t_ref.at[pl.ds(h1A * half, half), :], tA_loc, lsems.at[0])
        cpB = pltpu.make_async_copy(
            t_ref.at[pl.ds(breg + h1B * half, half), :], tB_loc, lsems.at[1])
        cpA.start()
        cpB.start()

        barrier_sem = pltpu.get_barrier_semaphore()
        for nbr in (xp, yp):
            pl.semaphore_signal(
                barrier_sem, inc=1,
                device_id=(nbr,), device_id_type=pl.DeviceIdType.MESH,
            )
        pl.semaphore_wait(barrier_sem, 2)

        def xchg(i, src, dst, partner):
            return pltpu.make_async_remote_copy(
                src_ref=src, dst_ref=dst,
                send_sem=ssems.at[i], recv_sem=rsems.at[i],
                device_id=(partner,), device_id_type=pl.DeviceIdType.MESH,
            )

        ra = xchg(0, t_ref.at[pl.ds((1 - h1A) * half, half), :], bufA1, xp)
        rb = xchg(1, t_ref.at[pl.ds(breg + (1 - h1B) * half, half), :],
                  bufB1, yp)
        ra.start()
        rb.start()
        cpA.wait()
        cpB.wait()
        ra.wait()
        bufA1[...] = bufA1[...] + tA_loc[...]
        rb.wait()
        bufB1[...] = bufB1[...] + tB_loc[...]

        ra = xchg(2, bufA1.at[pl.ds((1 - q2A) * quar, quar), :], bufA2, yp)
        rb = xchg(3, bufB1.at[pl.ds((1 - q2B) * quar, quar), :], bufB2, xp)
        ra.start()
        rb.start()
        ra.wait()
        fA[...] = f(bufA1[pl.ds(q2A * quar, quar), :] + bufA2[...])
        ga = xchg(4, fA, out_ref.at[pl.ds(gA, quar), :], yp)
        ga.start()
        cpA = pltpu.make_async_copy(fA, out_ref.at[pl.ds(gA, quar), :],
                                    lsems.at[2])
        cpA.start()
        rb.wait()
        fB[...] = f(bufB1[pl.ds(q2B * quar, quar), :] + bufB2[...])
        gb = xchg(5, fB, out_ref.at[pl.ds(gB, quar), :], xp)
        gb.start()
        cpB = pltpu.make_async_copy(fB, out_ref.at[pl.ds(gB, quar), :],
                                    lsems.at[3])
        cpB.start()
        ga.wait()
        gb.wait()
        cpA.wait()
        cpB.wait()

        ra = xchg(6, out_ref.at[pl.ds(h1A * half, half), :],
                  out_ref.at[pl.ds(h1A * half, half), :], xp)
        rb = xchg(7, out_ref.at[pl.ds(breg + h1B * half, half), :],
                  out_ref.at[pl.ds(breg + h1B * half, half), :], yp)
        ra.start()
        rb.start()
        ra.wait()
        rb.wait()

    return pl.pallas_call(
        body,
        out_shape=jax.ShapeDtypeStruct((m, n), t.dtype),
        in_specs=[pl.BlockSpec(memory_space=pl.ANY)],
        out_specs=pl.BlockSpec(memory_space=pl.ANY),
        scratch_shapes=[
            pltpu.VMEM((half, n), t.dtype),
            pltpu.VMEM((half, n), t.dtype),
            pltpu.VMEM((quar, n), t.dtype),
            pltpu.VMEM((quar, n), t.dtype),
            pltpu.VMEM((half, n), t.dtype),
            pltpu.VMEM((half, n), t.dtype),
            pltpu.VMEM((quar, n), t.dtype),
            pltpu.VMEM((quar, n), t.dtype),
            pltpu.SemaphoreType.DMA((8,)),
            pltpu.SemaphoreType.DMA((8,)),
            pltpu.SemaphoreType.DMA((4,)),
        ],
        compiler_params=pltpu.CompilerParams(collective_id=0),
    )(t)


# device time: 151063 ns/iter; 1.0130x vs baseline; 1.0130x over previous
import jax
import jax.numpy as jnp
from jax import lax
from jax.experimental import pallas as pl
from jax.experimental.pallas import tpu as pltpu

N_DEV = 4


def kernel(t):
    m, n = t.shape
    half = m // 4
    quar = m // 8
    breg = m // 2

    def f(s):
        r = jnp.maximum(s, 0.0)
        return jnp.tanh(s) * s * s + r * r * r

    def body(t_ref, out_ref, bufA1, bufB1, bufA2, bufB2, tA_loc, tB_loc,
             fA, fB, ssems, rsems, lsems):
        p = lax.axis_index("i")
        xp = p ^ 1
        yp = 3 - p

        h1A = jnp.where((p == 1) | (p == 2), 1, 0)
        q2A = jnp.where(p >= 2, 1, 0)
        h1B = q2A
        q2B = p % 2
        gA = (2 * h1A + q2A) * quar
        gB = breg + (2 * h1B + q2B) * quar

        cpA = pltpu.make_async_copy(
            t_ref.at[pl.ds(h1A * half, half), :], tA_loc, lsems.at[0])
        cpB = pltpu.make_async_copy(
            t_ref.at[pl.ds(breg + h1B * half, half), :], tB_loc, lsems.at[1])
        cpA.start()
        cpB.start()

        barrier_sem = pltpu.get_barrier_semaphore()
        for nbr in (xp, yp):
            pl.semaphore_signal(
                barrier_sem, inc=1,
                device_id=(nbr,), device_id_type=pl.DeviceIdType.MESH,
            )
        pl.semaphore_wait(barrier_sem, 2)

        def xchg(i, src, dst, partner):
            return pltpu.make_async_remote_copy(
                src_ref=src, dst_ref=dst,
                send_sem=ssems.at[i], recv_sem=rsems.at[i],
                device_id=(partner,), device_id_type=pl.DeviceIdType.MESH,
            )

        ra = xchg(0, t_ref.at[pl.ds((1 - h1A) * half, half), :], bufA1, xp)
        rb = xchg(1, t_ref.at[pl.ds(breg + (1 - h1B) * half, half), :],
                  bufB1, yp)
        ra.start()
        rb.start()
        cpA.wait()
        cpB.wait()
        ra.wait()
        sqA = (1 - q2A) * quar
        bufA1[pl.ds(sqA, quar), :] = (bufA1[pl.ds(sqA, quar), :]
                                      + tA_loc[pl.ds(sqA, quar), :])
        ra2 = xchg(2, bufA1.at[pl.ds(sqA, quar), :], bufA2, yp)
        ra2.start()
        rb.wait()
        sqB = (1 - q2B) * quar
        bufB1[pl.ds(sqB, quar), :] = (bufB1[pl.ds(sqB, quar), :]
                                      + tB_loc[pl.ds(sqB, quar), :])
        rb2 = xchg(3, bufB1.at[pl.ds(sqB, quar), :], bufB2, xp)
        rb2.start()

        ra2.wait()
        fA[...] = f(bufA1[pl.ds(q2A * quar, quar), :]
                    + tA_loc[pl.ds(q2A * quar, quar), :] + bufA2[...])
        ga1 = xchg(4, fA, out_ref.at[pl.ds(gA, quar), :], yp)
        ga1.start()
        ga2m = xchg(6, fA, out_ref.at[pl.ds(gA, quar), :], xp)
        ga2m.start()
        cpA2 = pltpu.make_async_copy(fA, out_ref.at[pl.ds(gA, quar), :],
                                     lsems.at[2])
        cpA2.start()
        rb2.wait()
        fB[...] = f(bufB1[pl.ds(q2B * quar, quar), :]
                    + tB_loc[pl.ds(q2B * quar, quar), :] + bufB2[...])
        gb1 = xchg(5, fB, out_ref.at[pl.ds(gB, quar), :], xp)
        gb1.start()
        gb2m = xchg(8, fB, out_ref.at[pl.ds(gB, quar), :], yp)
        gb2m.start()
        cpB2 = pltpu.make_async_copy(fB, out_ref.at[pl.ds(gB, quar), :],
                                     lsems.at[3])
        cpB2.start()

        gAyp = (2 * h1A + (1 - q2A)) * quar
        ga1.wait_recv()
        ga2r = xchg(7, out_ref.at[pl.ds(gAyp, quar), :],
                    out_ref.at[pl.ds(gAyp, quar), :], xp)
        ga2r.start()
        gBxp = breg + (2 * h1B + (1 - q2B)) * quar
        gb1.wait_recv()
        gb2r = xchg(9, out_ref.at[pl.ds(gBxp, quar), :],
                    out_ref.at[pl.ds(gBxp, quar), :], yp)
        gb2r.start()

        ga1.wait_send()
        gb1.wait_send()
        ga2m.wait()
        gb2m.wait()
        ga2r.wait()
        gb2r.wait()
        cpA2.wait()
        cpB2.wait()

    return pl.pallas_call(
        body,
        out_shape=jax.ShapeDtypeStruct((m, n), t.dtype),
        in_specs=[pl.BlockSpec(memory_space=pl.ANY)],
        out_specs=pl.BlockSpec(memory_space=pl.ANY),
        scratch_shapes=[
            pltpu.VMEM((half, n), t.dtype),
            pltpu.VMEM((half, n), t.dtype),
            pltpu.VMEM((quar, n), t.dtype),
            pltpu.VMEM((quar, n), t.dtype),
            pltpu.VMEM((half, n), t.dtype),
            pltpu.VMEM((half, n), t.dtype),
            pltpu.VMEM((quar, n), t.dtype),
            pltpu.VMEM((quar, n), t.dtype),
            pltpu.SemaphoreType.DMA((10,)),
            pltpu.SemaphoreType.DMA((10,)),
            pltpu.SemaphoreType.DMA((4,)),
        ],
        compiler_params=pltpu.CompilerParams(collective_id=0),
    )(t)


# device time: 146543 ns/iter; 1.0442x vs baseline; 1.0308x over previous
import jax
import jax.numpy as jnp
from jax import lax
from jax.experimental import pallas as pl
from jax.experimental.pallas import tpu as pltpu

N_DEV = 4
NG = 2


def kernel(t):
    m, n = t.shape
    half = m // 4
    quar = m // 8
    breg = m // 2
    cw = n // NG

    def f(s):
        r = jnp.maximum(s, 0.0)
        return jnp.tanh(s) * s * s + r * r * r

    def body(t_ref, out_ref, bufA1, bufB1, bufA2, bufB2, tA_loc, tB_loc,
             fA, fB, ssems, rsems, lsems):
        p = lax.axis_index("i")
        xp = p ^ 1
        yp = 3 - p

        h1A = jnp.where((p == 1) | (p == 2), 1, 0)
        q2A = jnp.where(p >= 2, 1, 0)
        h1B = q2A
        q2B = p % 2
        gA = (2 * h1A + q2A) * quar
        gB = breg + (2 * h1B + q2B) * quar
        gAyp = (2 * h1A + (1 - q2A)) * quar
        gBxp = breg + (2 * h1B + (1 - q2B)) * quar
        sqA = (1 - q2A) * quar
        sqB = (1 - q2B) * quar

        def cols(g):
            return pl.ds(g * cw, cw)

        cpA = pltpu.make_async_copy(
            t_ref.at[pl.ds(h1A * half, half), :], tA_loc, lsems.at[0])
        cpB = pltpu.make_async_copy(
            t_ref.at[pl.ds(breg + h1B * half, half), :], tB_loc, lsems.at[1])
        cpA.start()
        cpB.start()

        barrier_sem = pltpu.get_barrier_semaphore()
        for nbr in (xp, yp):
            pl.semaphore_signal(
                barrier_sem, inc=1,
                device_id=(nbr,), device_id_type=pl.DeviceIdType.MESH,
            )
        pl.semaphore_wait(barrier_sem, 2)

        def xchg(g, i, src, dst, partner):
            return pltpu.make_async_remote_copy(
                src_ref=src, dst_ref=dst,
                send_sem=ssems.at[10 * g + i], recv_sem=rsems.at[10 * g + i],
                device_id=(partner,), device_id_type=pl.DeviceIdType.MESH,
            )

        ra1, rb1 = [], []
        for g in range(NG):
            ra = xchg(g, 0, t_ref.at[pl.ds((1 - h1A) * half, half), cols(g)],
                      bufA1.at[g], xp)
            rb = xchg(g, 1,
                      t_ref.at[pl.ds(breg + (1 - h1B) * half, half), cols(g)],
                      bufB1.at[g], yp)
            ra.start()
            rb.start()
            ra1.append(ra)
            rb1.append(rb)
        cpA.wait()
        cpB.wait()

        ra2, rb2 = [], []
        for g in range(NG):
            ra1[g].wait()
            bufA1[g, pl.ds(sqA, quar), :] = (
                bufA1[g, pl.ds(sqA, quar), :]
                + tA_loc[pl.ds(sqA, quar), cols(g)])
            ra = xchg(g, 2, bufA1.at[g, pl.ds(sqA, quar), :],
                      bufA2.at[g], yp)
            ra.start()
            ra2.append(ra)
            rb1[g].wait()
            bufB1[g, pl.ds(sqB, quar), :] = (
                bufB1[g, pl.ds(sqB, quar), :]
                + tB_loc[pl.ds(sqB, quar), cols(g)])
            rb = xchg(g, 3, bufB1.at[g, pl.ds(sqB, quar), :],
                      bufB2.at[g], xp)
            rb.start()
            rb2.append(rb)

        ga1, ga2m, gb1, gb2m, cps = [], [], [], [], []
        for g in range(NG):
            ra2[g].wait()
            fA[g] = f(bufA1[g, pl.ds(q2A * quar, quar), :]
                      + tA_loc[pl.ds(q2A * quar, quar), cols(g)]
                      + bufA2[g])
            x1 = xchg(g, 4, fA.at[g], out_ref.at[pl.ds(gA, quar), cols(g)], yp)
            x1.start()
            x2 = xchg(g, 6, fA.at[g], out_ref.at[pl.ds(gA, quar), cols(g)], xp)
            x2.start()
            cp = pltpu.make_async_copy(
                fA.at[g], out_ref.at[pl.ds(gA, quar), cols(g)],
                lsems.at[2 + g])
            cp.start()
            ga1.append(x1)
            ga2m.append(x2)
            cps.append(cp)
            rb2[g].wait()
            fB[g] = f(bufB1[g, pl.ds(q2B * quar, quar), :]
                      + tB_loc[pl.ds(q2B * quar, quar), cols(g)]
                      + bufB2[g])
            x1 = xchg(g, 5, fB.at[g], out_ref.at[pl.ds(gB, quar), cols(g)], xp)
            x1.start()
            x2 = xchg(g, 8, fB.at[g], out_ref.at[pl.ds(gB, quar), cols(g)], yp)
            x2.start()
            cp = pltpu.make_async_copy(
                fB.at[g], out_ref.at[pl.ds(gB, quar), cols(g)],
                lsems.at[4 + g])
            cp.start()
            gb1.append(x1)
            gb2m.append(x2)
            cps.append(cp)

        ga2r, gb2r = [], []
        for g in range(NG):
            ga1[g].wait_recv()
            x = xchg(g, 7, out_ref.at[pl.ds(gAyp, quar), cols(g)],
                     out_ref.at[pl.ds(gAyp, quar), cols(g)], xp)
            x.start()
            ga2r.append(x)
            gb1[g].wait_recv()
            x = xchg(g, 9, out_ref.at[pl.ds(gBxp, quar), cols(g)],
                     out_ref.at[pl.ds(gBxp, quar), cols(g)], yp)
            x.start()
            gb2r.append(x)

        for g in range(NG):
            ga1[g].wait_send()
            gb1[g].wait_send()
            ga2m[g].wait()
            gb2m[g].wait()
            ga2r[g].wait()
            gb2r[g].wait()
        for cp in cps:
            cp.wait()

    return pl.pallas_call(
        body,
        out_shape=jax.ShapeDtypeStruct((m, n), t.dtype),
        in_specs=[pl.BlockSpec(memory_space=pl.ANY)],
        out_specs=pl.BlockSpec(memory_space=pl.ANY),
        scratch_shapes=[
            pltpu.VMEM((NG, half, cw), t.dtype),
            pltpu.VMEM((NG, half, cw), t.dtype),
            pltpu.VMEM((NG, quar, cw), t.dtype),
            pltpu.VMEM((NG, quar, cw), t.dtype),
            pltpu.VMEM((half, n), t.dtype),
            pltpu.VMEM((half, n), t.dtype),
            pltpu.VMEM((NG, quar, cw), t.dtype),
            pltpu.VMEM((NG, quar, cw), t.dtype),
            pltpu.SemaphoreType.DMA((10 * NG,)),
            pltpu.SemaphoreType.DMA((10 * NG,)),
            pltpu.SemaphoreType.DMA((2 + 2 * NG,)),
        ],
        compiler_params=pltpu.CompilerParams(collective_id=0),
    )(t)


# device time: 146541 ns/iter; 1.0442x vs baseline; 1.0000x over previous
import jax
import jax.numpy as jnp
from jax import lax
from jax.experimental import pallas as pl
from jax.experimental.pallas import tpu as pltpu

N_DEV = 4
NG = 2


def kernel(t):
    m, n = t.shape
    half = m // 4
    quar = m // 8
    breg = m // 2
    cw = n // NG

    def f(s):
        r = jnp.maximum(s, 0.0)
        return jnp.tanh(s) * s * s + r * r * r

    def body(t_ref, out_ref, bufA1, bufB1, bufA2, bufB2, tA_loc, tB_loc,
             fA, fB, ssems, rsems, lsems):
        p = lax.axis_index("i")
        xp = p ^ 1
        yp = 3 - p

        h1A = jnp.where((p == 1) | (p == 2), 1, 0)
        q2A = jnp.where(p >= 2, 1, 0)
        h1B = q2A
        q2B = p % 2
        gA = (2 * h1A + q2A) * quar
        gB = breg + (2 * h1B + q2B) * quar
        gAyp = (2 * h1A + (1 - q2A)) * quar
        gBxp = breg + (2 * h1B + (1 - q2B)) * quar
        sqA = (1 - q2A) * quar
        sqB = (1 - q2B) * quar

        def cols(g):
            return pl.ds(g * cw, cw)

        cpA = pltpu.make_async_copy(
            t_ref.at[pl.ds(h1A * half, half), :], tA_loc, lsems.at[0])
        cpB = pltpu.make_async_copy(
            t_ref.at[pl.ds(breg + h1B * half, half), :], tB_loc, lsems.at[1])
        cpA.start()
        cpB.start()

        barrier_sem = pltpu.get_barrier_semaphore()
        for nbr in (xp, yp):
            pl.semaphore_signal(
                barrier_sem, inc=1,
                device_id=(nbr,), device_id_type=pl.DeviceIdType.MESH,
            )
        pl.semaphore_wait(barrier_sem, 2)

        def xchg(g, i, src, dst, partner):
            return pltpu.make_async_remote_copy(
                src_ref=src, dst_ref=dst,
                send_sem=ssems.at[10 * g + i], recv_sem=rsems.at[10 * g + i],
                device_id=(partner,), device_id_type=pl.DeviceIdType.MESH,
            )

        ra1, rb1 = [], []
        for g in range(NG):
            ra = xchg(g, 0, t_ref.at[pl.ds((1 - h1A) * half, half), cols(g)],
                      bufA1.at[g], xp)
            rb = xchg(g, 1,
                      t_ref.at[pl.ds(breg + (1 - h1B) * half, half), cols(g)],
                      bufB1.at[g], yp)
            ra.start()
            rb.start()
            ra1.append(ra)
            rb1.append(rb)
        cpA.wait()
        cpB.wait()

        ra2, rb2 = [], []
        for g in range(NG):
            ra1[g].wait()
            bufA1[g, pl.ds(sqA, quar), :] = (
                bufA1[g, pl.ds(sqA, quar), :]
                + tA_loc[pl.ds(sqA, quar), cols(g)])
            ra = xchg(g, 2, bufA1.at[g, pl.ds(sqA, quar), :],
                      bufA2.at[g], yp)
            ra.start()
            ra2.append(ra)
            rb1[g].wait()
            bufB1[g, pl.ds(sqB, quar), :] = (
                bufB1[g, pl.ds(sqB, quar), :]
                + tB_loc[pl.ds(sqB, quar), cols(g)])
            rb = xchg(g, 3, bufB1.at[g, pl.ds(sqB, quar), :],
                      bufB2.at[g], xp)
            rb.start()
            rb2.append(rb)

        ga1, ga2m, gb1, gb2m, cps = [], [], [], [], []
        for g in range(NG):
            ra2[g].wait()
            fA[g] = f(bufA1[g, pl.ds(q2A * quar, quar), :]
                      + tA_loc[pl.ds(q2A * quar, quar), cols(g)]
                      + bufA2[g])
            x1 = xchg(g, 4, fA.at[g], out_ref.at[pl.ds(gA, quar), cols(g)], yp)
            x1.start()
            x2 = xchg(g, 6, fA.at[g], out_ref.at[pl.ds(gA, quar), cols(g)], xp)
            x2.start()
            cp = pltpu.make_async_copy(
                fA.at[g], out_ref.at[pl.ds(gA, quar), cols(g)],
                lsems.at[2 + g])
            cp.start()
            ga1.append(x1)
            ga2m.append(x2)
            cps.append(cp)
            rb2[g].wait()
            fB[g] = f(bufB1[g, pl.ds(q2B * quar, quar), :]
                      + tB_loc[pl.ds(q2B * quar, quar), cols(g)]
                      + bufB2[g])
            x1 = xchg(g, 5, fB.at[g], out_ref.at[pl.ds(gB, quar), cols(g)], xp)
            x1.start()
            x2 = xchg(g, 8, fB.at[g], out_ref.at[pl.ds(gB, quar), cols(g)], yp)
            x2.start()
            cp = pltpu.make_async_copy(
                fB.at[g], out_ref.at[pl.ds(gB, quar), cols(g)],
                lsems.at[2 + NG + g])
            cp.start()
            gb1.append(x1)
            gb2m.append(x2)
            cps.append(cp)

        ga2r, gb2r = [], []
        for g in range(NG):
            ga1[g].wait_recv()
            x = xchg(g, 7, out_ref.at[pl.ds(gAyp, quar), cols(g)],
                     out_ref.at[pl.ds(gAyp, quar), cols(g)], xp)
            x.start()
            ga2r.append(x)
            gb1[g].wait_recv()
            x = xchg(g, 9, out_ref.at[pl.ds(gBxp, quar), cols(g)],
                     out_ref.at[pl.ds(gBxp, quar), cols(g)], yp)
            x.start()
            gb2r.append(x)

        for g in range(NG):
            ga1[g].wait_send()
            gb1[g].wait_send()
            ga2m[g].wait()
            gb2m[g].wait()
            ga2r[g].wait()
            gb2r[g].wait()
        for cp in cps:
            cp.wait()

    return pl.pallas_call(
        body,
        out_shape=jax.ShapeDtypeStruct((m, n), t.dtype),
        in_specs=[pl.BlockSpec(memory_space=pl.ANY)],
        out_specs=pl.BlockSpec(memory_space=pl.ANY),
        scratch_shapes=[
            pltpu.VMEM((NG, half, cw), t.dtype),
            pltpu.VMEM((NG, half, cw), t.dtype),
            pltpu.VMEM((NG, quar, cw), t.dtype),
            pltpu.VMEM((NG, quar, cw), t.dtype),
            pltpu.VMEM((half, n), t.dtype),
            pltpu.VMEM((half, n), t.dtype),
            pltpu.VMEM((NG, quar, cw), t.dtype),
            pltpu.VMEM((NG, quar, cw), t.dtype),
            pltpu.SemaphoreType.DMA((10 * NG,)),
            pltpu.SemaphoreType.DMA((10 * NG,)),
            pltpu.SemaphoreType.DMA((2 + 2 * NG,)),
        ],
        compiler_params=pltpu.CompilerParams(collective_id=0),
    )(t)
